# Initial kernel scaffold; baseline (speedup 1.0000x reference)
import jax
import jax.numpy as jnp
from jax.experimental import pallas as pl


def kernel(pred, target):
    raise NotImplementedError("write your pallas kernel here")


# speedup vs baseline: 1.0877x; 1.0877x over previous
import jax
import jax.numpy as jnp
from jax import lax
from jax.experimental import pallas as pl
from jax.experimental.pallas import tpu as pltpu

_ROWS = 4 * 4096
_D = 1024
_K = _ROWS // 2
_BLK = 2048
_NBLK = _ROWS // _BLK


def _topk_l1_body(pred_ref, target_ref, out_ref, loss_ref):
    i = pl.program_id(0)
    s = jnp.sum(jnp.abs(pred_ref[...] - target_ref[...]), axis=1)
    loss_ref[i, :] = s

    @pl.when(i == _NBLK - 1)
    def _():
        v = loss_ref[...]
        bits = lax.bitcast_convert_type(v, jnp.int32)

        def step(_, carry):
            lo, hi = carry
            mid = lo + (hi - lo + 1) // 2
            cnt = jnp.sum((bits >= mid).astype(jnp.int32))
            ok = cnt >= _K
            return jnp.where(ok, mid, lo), jnp.where(ok, hi, mid - 1)

        lo, _hi = lax.fori_loop(
            0, 31, step, (jnp.int32(0), jnp.int32(0x7F7FFFFF)))
        t = lax.bitcast_convert_type(lo, jnp.float32)
        gt = bits > lo
        m = jnp.sum(gt.astype(jnp.int32)).astype(jnp.float32)
        sum_gt = jnp.sum(jnp.where(gt, v, 0.0))
        total = sum_gt + (jnp.float32(_K) - m) * t
        out_ref[...] = jnp.full((1, 1), total / jnp.float32(_K), jnp.float32)


def kernel(pred, target):
    p = pred.reshape(_ROWS, _D)
    t = target.reshape(_ROWS, _D)
    out = pl.pallas_call(
        _topk_l1_body,
        grid=(_NBLK,),
        in_specs=[
            pl.BlockSpec((_BLK, _D), lambda i: (i, 0)),
            pl.BlockSpec((_BLK, _D), lambda i: (i, 0)),
        ],
        out_specs=pl.BlockSpec((1, 1), lambda i: (0, 0)),
        out_shape=jax.ShapeDtypeStruct((1, 1), jnp.float32),
        scratch_shapes=[pltpu.VMEM((_NBLK, _BLK), jnp.float32)],
    )(p, t)
    return out[0, 0]
